# Initial kernel scaffold; baseline (speedup 1.0000x reference)
import jax
import jax.numpy as jnp
from jax.experimental import pallas as pl


def kernel(x, edge_index, batch, W1, b1, W2, b2, gamma, beta, ow1, ob1, ow2, ob2):
    raise NotImplementedError("write your pallas kernel here")


# speedup vs baseline: 24.1684x; 24.1684x over previous
import functools

import jax
import jax.numpy as jnp
from jax import lax
from jax.experimental import pallas as pl
from jax.experimental.pallas import tpu as pltpu
from jax.experimental.pallas import tpu_sc as plsc

N = 10000
G = 64
NC, NS = 2, 16
CH = 128
CHUNKS = 80
E_PAD = NC * NS * CHUNKS * CH
ROWS_PER_TILE = N // NS
RB = 2000
F1 = 48
F2 = 16


def _sc_mesh():
    return plsc.VectorSubcoreMesh(core_axis_name="c", subcore_axis_name="s")


_SC_PARAMS = pltpu.CompilerParams(use_tc_tiling_on_sc=False)


def _deg_kernel(zeros_hbm, ones_hbm, dst_hbm):
    @functools.partial(
        pl.kernel,
        out_type=jax.ShapeDtypeStruct((NC, N, 16), jnp.float32),
        mesh=_sc_mesh(),
        compiler_params=_SC_PARAMS,
        scratch_types=[
            pltpu.VMEM((CHUNKS, CH), jnp.int32),
            pltpu.VMEM((CH, 16), jnp.float32),
            pltpu.VMEM_SHARED((N + 16, 16), jnp.float32),
        ],
    )
    def k(zeros_h, ones_h, dst_h, out_h, dst_v, ones_v, accum):
        c = lax.axis_index("c")
        s = lax.axis_index("s")
        row0 = s * ROWS_PER_TILE
        pltpu.sync_copy(dst_h.at[c, s], dst_v)
        pltpu.sync_copy(ones_h, ones_v)
        pltpu.sync_copy(zeros_h.at[pl.ds(row0, ROWS_PER_TILE)],
                        accum.at[pl.ds(row0, ROWS_PER_TILE)])
        plsc.subcore_barrier()

        def body(j, carry):
            pltpu.sync_copy(ones_v, accum.at[dst_v.at[j]], add=True)
            return carry

        lax.fori_loop(0, CHUNKS, body, 0)
        plsc.subcore_barrier()
        pltpu.sync_copy(accum.at[pl.ds(row0, ROWS_PER_TILE)],
                        out_h.at[c, pl.ds(row0, ROWS_PER_TILE)])

    return k(zeros_hbm, ones_hbm, dst_hbm)


def _edge_pass(table_hbm, src_hbm, dst_hbm, F):
    @functools.partial(
        pl.kernel,
        out_type=jax.ShapeDtypeStruct((NC, N, F), jnp.float32),
        mesh=_sc_mesh(),
        compiler_params=_SC_PARAMS,
        scratch_types=[
            pltpu.VMEM((CHUNKS, CH), jnp.int32),
            pltpu.VMEM((CHUNKS, CH), jnp.int32),
            pltpu.VMEM((CH, F), jnp.float32),
            pltpu.VMEM_SHARED((N + 16, F), jnp.float32),
            pltpu.SemaphoreType.DMA,
        ],
    )
    def k(table_h, src_h, dst_h, out_h, src_v, dst_v, rows_v, accum, gsem):
        c = lax.axis_index("c")
        s = lax.axis_index("s")
        row0 = s * ROWS_PER_TILE
        pltpu.sync_copy(src_h.at[c, s], src_v)
        pltpu.sync_copy(dst_h.at[c, s], dst_v)
        pltpu.sync_copy(table_h.at[pl.ds(row0, ROWS_PER_TILE)],
                        accum.at[pl.ds(row0, ROWS_PER_TILE)])
        plsc.subcore_barrier()

        def body(j, carry):
            pltpu.async_copy(table_h.at[src_v.at[j]], rows_v, gsem).wait()
            pltpu.sync_copy(rows_v, accum.at[dst_v.at[j]], add=True)
            return carry

        lax.fori_loop(0, CHUNKS, body, 0)
        plsc.subcore_barrier()
        pltpu.sync_copy(accum.at[pl.ds(row0, ROWS_PER_TILE)],
                        out_h.at[c, pl.ds(row0, ROWS_PER_TILE)])

    return k(table_hbm, src_hbm, dst_hbm)


def _tc_b_body(degp_ref, x_ref, w1_ref, y1_ref, dinv_ref):
    deg = degp_ref[0, :, 0:1] + degp_ref[1, :, 0:1] + 1.0
    dinv = lax.rsqrt(deg)
    xw = jnp.dot(x_ref[...], w1_ref[...], preferred_element_type=jnp.float32)
    y = dinv * xw
    y1_ref[...] = jnp.concatenate(
        [y, jnp.zeros((y.shape[0], F1 - y.shape[1]), jnp.float32)], axis=1)
    dinv_ref[...] = dinv


def _tc_b(degp, x, W1):
    grid = N // RB
    return pl.pallas_call(
        _tc_b_body,
        grid=(grid,),
        in_specs=[
            pl.BlockSpec((NC, RB, 16), lambda i: (0, i, 0)),
            pl.BlockSpec((RB, 128), lambda i: (i, 0)),
            pl.BlockSpec((128, 36), lambda i: (0, 0)),
        ],
        out_specs=[
            pl.BlockSpec((RB, F1), lambda i: (i, 0)),
            pl.BlockSpec((RB, 1), lambda i: (i, 0)),
        ],
        out_shape=[
            jax.ShapeDtypeStruct((N, F1), jnp.float32),
            jax.ShapeDtypeStruct((N, 1), jnp.float32),
        ],
    )(degp, x, W1)


def _tc_d_body(p_ref, y1_ref, dinv_ref, b1_ref, w2_ref, y2_ref):
    tot = p_ref[0] + p_ref[1] - y1_ref[...]
    h1 = jnp.maximum(tot * dinv_ref[...] + b1_ref[...], 0.0)
    xw2 = jnp.dot(h1, w2_ref[...], preferred_element_type=jnp.float32)
    y2_ref[...] = dinv_ref[...] * xw2


def _tc_d(p1, y1, dinv, b1p, W2p):
    grid = N // RB
    return pl.pallas_call(
        _tc_d_body,
        grid=(grid,),
        in_specs=[
            pl.BlockSpec((NC, RB, F1), lambda i: (0, i, 0)),
            pl.BlockSpec((RB, F1), lambda i: (i, 0)),
            pl.BlockSpec((RB, 1), lambda i: (i, 0)),
            pl.BlockSpec((1, F1), lambda i: (0, 0)),
            pl.BlockSpec((F1, F2), lambda i: (0, 0)),
        ],
        out_specs=pl.BlockSpec((RB, F2), lambda i: (i, 0)),
        out_shape=jax.ShapeDtypeStruct((N, F2), jnp.float32),
    )(p1, y1, dinv, b1p, W2p)


def _tc_f_body(p_ref, y2_ref, dinv_ref, b2_ref, batch_ref, gamma_ref,
               beta_ref, ow1_ref, ob1_ref, ow2_ref, ob2_ref,
               o_ref, h_ref, acc):
    i = pl.program_id(0)

    @pl.when(i == 0)
    def _():
        acc[...] = jnp.zeros_like(acc)

    tot = p_ref[0] + p_ref[1] - y2_ref[...]
    h2 = jnp.maximum(tot * dinv_ref[...] + b2_ref[...], 0.0)
    bt = batch_ref[0]
    gids = lax.broadcasted_iota(jnp.int32, (G, RB), 0)
    onehot = (gids == bt).astype(jnp.float32)
    acc[...] += jnp.dot(onehot, h2, preferred_element_type=jnp.float32)

    @pl.when(i == pl.num_programs(0) - 1)
    def _():
        pooled = acc[...]
        h_ref[...] = pooled
        mean = jnp.mean(pooled, axis=0, keepdims=True)
        var = jnp.mean((pooled - mean) ** 2, axis=0, keepdims=True)
        xb = ((pooled - mean) * lax.rsqrt(var + 1e-5) * gamma_ref[...]
              + beta_ref[...])
        t = jnp.maximum(
            jnp.dot(xb, ow1_ref[...], preferred_element_type=jnp.float32)
            + ob1_ref[...], 0.0)
        o_ref[...] = (jnp.dot(t, ow2_ref[...],
                              preferred_element_type=jnp.float32)
                      + ob2_ref[...])


def _tc_f(p2, y2, dinv, b2, batch2d, gamma, beta, ow1, ob1, ow2, ob2):
    grid = N // RB
    return pl.pallas_call(
        _tc_f_body,
        grid=(grid,),
        in_specs=[
            pl.BlockSpec((NC, RB, F2), lambda i: (0, i, 0)),
            pl.BlockSpec((RB, F2), lambda i: (i, 0)),
            pl.BlockSpec((RB, 1), lambda i: (i, 0)),
            pl.BlockSpec((1, F2), lambda i: (0, 0)),
            pl.BlockSpec((1, 1, RB), lambda i: (i, 0, 0)),
            pl.BlockSpec((1, F2), lambda i: (0, 0)),
            pl.BlockSpec((1, F2), lambda i: (0, 0)),
            pl.BlockSpec((F2, 24), lambda i: (0, 0)),
            pl.BlockSpec((1, 24), lambda i: (0, 0)),
            pl.BlockSpec((24, 1), lambda i: (0, 0)),
            pl.BlockSpec((1, 1), lambda i: (0, 0)),
        ],
        out_specs=[
            pl.BlockSpec((G, 1), lambda i: (0, 0)),
            pl.BlockSpec((G, F2), lambda i: (0, 0)),
        ],
        out_shape=[
            jax.ShapeDtypeStruct((G, 1), jnp.float32),
            jax.ShapeDtypeStruct((G, F2), jnp.float32),
        ],
        scratch_shapes=[pltpu.VMEM((G, F2), jnp.float32)],
    )(p2, y2, dinv, b2, batch2d, gamma, beta, ow1, ob1, ow2, ob2)


def kernel(x, edge_index, batch, W1, b1, W2, b2, gamma, beta, ow1, ob1, ow2,
           ob2):
    src = edge_index[0].astype(jnp.int32)
    dst = edge_index[1].astype(jnp.int32)
    e = src.shape[0]
    pad = E_PAD - e
    src_p = jnp.concatenate([src, jnp.zeros((pad,), jnp.int32)])
    dst_p = jnp.concatenate([dst, jnp.full((pad,), N, jnp.int32)])
    src_p = src_p.reshape(NC, NS, CHUNKS, CH)
    dst_p = dst_p.reshape(NC, NS, CHUNKS, CH)

    zeros16 = jnp.zeros((N, 16), jnp.float32)
    ones_blk = jnp.ones((CH, 16), jnp.float32)
    degp = _deg_kernel(zeros16, ones_blk, dst_p)

    y1, dinv = _tc_b(degp, x, W1)
    p1 = _edge_pass(y1, src_p, dst_p, F1)

    b1p = jnp.pad(b1, (0, F1 - b1.shape[0])).reshape(1, F1)
    W2p = jnp.pad(W2, ((0, F1 - W2.shape[0]), (0, 0)))
    y2 = _tc_d(p1, y1, dinv, b1p, W2p)
    p2 = _edge_pass(y2, src_p, dst_p, F2)

    batch2d = batch.astype(jnp.int32).reshape(N // RB, 1, RB)
    o, h = _tc_f(p2, y2, dinv, b2.reshape(1, F2), batch2d,
                 gamma.reshape(1, F2), beta.reshape(1, F2), ow1,
                 ob1.reshape(1, 24), ow2, ob2.reshape(1, 1))
    return (o, h)


# speedup vs baseline: 30.2185x; 1.2503x over previous
import functools

import jax
import jax.numpy as jnp
from jax import lax
from jax.experimental import pallas as pl
from jax.experimental.pallas import tpu as pltpu
from jax.experimental.pallas import tpu_sc as plsc

N = 10000
G = 64
NC, NS = 2, 16
CH = 128
CHUNKS = 80
K = 8
E_PAD = NC * NS * CHUNKS * CH
ROWS_PER_TILE = N // NS
RB = 2000
F1 = 48
F2 = 16


def _sc_mesh():
    return plsc.VectorSubcoreMesh(core_axis_name="c", subcore_axis_name="s")


_SC_PARAMS = pltpu.CompilerParams(use_tc_tiling_on_sc=False)


def _deg_kernel(zeros_hbm, ones_hbm, dst_hbm):
    @functools.partial(
        pl.kernel,
        out_type=jax.ShapeDtypeStruct((NC, N, 16), jnp.float32),
        mesh=_sc_mesh(),
        compiler_params=_SC_PARAMS,
        scratch_types=[
            pltpu.VMEM((CHUNKS, CH), jnp.int32),
            pltpu.VMEM((CH, 16), jnp.float32),
            pltpu.VMEM_SHARED((N + 16, 16), jnp.float32),
            pltpu.SemaphoreType.DMA,
        ],
    )
    def k(zeros_h, ones_h, dst_h, out_h, dst_v, ones_v, accum, ssem):
        c = lax.axis_index("c")
        s = lax.axis_index("s")
        row0 = s * ROWS_PER_TILE
        pltpu.sync_copy(dst_h.at[c, s], dst_v)
        pltpu.sync_copy(ones_h, ones_v)
        pltpu.sync_copy(zeros_h.at[pl.ds(row0, ROWS_PER_TILE)],
                        accum.at[pl.ds(row0, ROWS_PER_TILE)])
        plsc.subcore_barrier()

        def body(g, carry):
            descs = [pltpu.async_copy(ones_v, accum.at[dst_v.at[g * K + b]],
                                      ssem, add=True) for b in range(K)]
            for d in descs:
                d.wait()
            return carry

        lax.fori_loop(0, CHUNKS // K, body, 0)
        plsc.subcore_barrier()
        pltpu.sync_copy(accum.at[pl.ds(row0, ROWS_PER_TILE)],
                        out_h.at[c, pl.ds(row0, ROWS_PER_TILE)])

    return k(zeros_hbm, ones_hbm, dst_hbm)


def _edge_pass(table_hbm, src_hbm, dst_hbm, F):
    @functools.partial(
        pl.kernel,
        out_type=jax.ShapeDtypeStruct((NC, N, F), jnp.float32),
        mesh=_sc_mesh(),
        compiler_params=_SC_PARAMS,
        scratch_types=[
            pltpu.VMEM((CHUNKS, CH), jnp.int32),
            pltpu.VMEM((CHUNKS, CH), jnp.int32),
            pltpu.VMEM((K, CH, F), jnp.float32),
            pltpu.VMEM_SHARED((N + 16, F), jnp.float32),
            pltpu.SemaphoreType.DMA((K,)),
            pltpu.SemaphoreType.DMA((K,)),
        ],
    )
    def k(table_h, src_h, dst_h, out_h, src_v, dst_v, rows_v, accum, gsem,
          ssem):
        c = lax.axis_index("c")
        s = lax.axis_index("s")
        row0 = s * ROWS_PER_TILE
        pltpu.sync_copy(src_h.at[c, s], src_v)
        pltpu.sync_copy(dst_h.at[c, s], dst_v)
        pltpu.sync_copy(table_h.at[pl.ds(row0, ROWS_PER_TILE)],
                        accum.at[pl.ds(row0, ROWS_PER_TILE)])
        plsc.subcore_barrier()

        ng = CHUNKS // K

        def fire_gather(g, b):
            return pltpu.async_copy(table_h.at[src_v.at[g * K + b]],
                                    rows_v.at[b], gsem.at[b])

        def gather_desc(g, b):
            return pltpu.make_async_copy(table_h.at[src_v.at[g * K + b]],
                                         rows_v.at[b], gsem.at[b])

        def fire_scatter(g, b):
            return pltpu.async_copy(rows_v.at[b],
                                    accum.at[dst_v.at[g * K + b]],
                                    ssem.at[b], add=True)

        def scatter_desc(g, b):
            return pltpu.make_async_copy(rows_v.at[b],
                                         accum.at[dst_v.at[g * K + b]],
                                         ssem.at[b])

        for b in range(K):
            fire_gather(0, b)

        def body(g, carry):
            for b in range(K):
                gather_desc(g, b).wait()
                fire_scatter(g, b)
            for b in range(K):
                scatter_desc(g, b).wait()
                fire_gather(g + 1, b)
            return carry

        lax.fori_loop(0, ng - 1, body, 0)
        for b in range(K):
            gather_desc(ng - 1, b).wait()
            fire_scatter(ng - 1, b)
        for b in range(K):
            scatter_desc(ng - 1, b).wait()
        plsc.subcore_barrier()
        pltpu.sync_copy(accum.at[pl.ds(row0, ROWS_PER_TILE)],
                        out_h.at[c, pl.ds(row0, ROWS_PER_TILE)])

    return k(table_hbm, src_hbm, dst_hbm)


def _tc_b_body(degp_ref, x_ref, w1_ref, y1_ref, dinv_ref):
    deg = degp_ref[0, :, 0:1] + degp_ref[1, :, 0:1] + 1.0
    dinv = lax.rsqrt(deg)
    xw = jnp.dot(x_ref[...], w1_ref[...], preferred_element_type=jnp.float32, precision=lax.Precision.HIGHEST)
    y = dinv * xw
    y1_ref[...] = jnp.concatenate(
        [y, jnp.zeros((y.shape[0], F1 - y.shape[1]), jnp.float32)], axis=1)
    dinv_ref[...] = dinv


def _tc_b(degp, x, W1):
    grid = N // RB
    return pl.pallas_call(
        _tc_b_body,
        grid=(grid,),
        in_specs=[
            pl.BlockSpec((NC, RB, 16), lambda i: (0, i, 0)),
            pl.BlockSpec((RB, 128), lambda i: (i, 0)),
            pl.BlockSpec((128, 36), lambda i: (0, 0)),
        ],
        out_specs=[
            pl.BlockSpec((RB, F1), lambda i: (i, 0)),
            pl.BlockSpec((RB, 1), lambda i: (i, 0)),
        ],
        out_shape=[
            jax.ShapeDtypeStruct((N, F1), jnp.float32),
            jax.ShapeDtypeStruct((N, 1), jnp.float32),
        ],
    )(degp, x, W1)


def _tc_d_body(p_ref, y1_ref, dinv_ref, b1_ref, w2_ref, y2_ref):
    tot = p_ref[0] + p_ref[1] - y1_ref[...]
    h1 = jnp.maximum(tot * dinv_ref[...] + b1_ref[...], 0.0)
    xw2 = jnp.dot(h1, w2_ref[...], preferred_element_type=jnp.float32, precision=lax.Precision.HIGHEST)
    y2_ref[...] = dinv_ref[...] * xw2


def _tc_d(p1, y1, dinv, b1p, W2p):
    grid = N // RB
    return pl.pallas_call(
        _tc_d_body,
        grid=(grid,),
        in_specs=[
            pl.BlockSpec((NC, RB, F1), lambda i: (0, i, 0)),
            pl.BlockSpec((RB, F1), lambda i: (i, 0)),
            pl.BlockSpec((RB, 1), lambda i: (i, 0)),
            pl.BlockSpec((1, F1), lambda i: (0, 0)),
            pl.BlockSpec((F1, F2), lambda i: (0, 0)),
        ],
        out_specs=pl.BlockSpec((RB, F2), lambda i: (i, 0)),
        out_shape=jax.ShapeDtypeStruct((N, F2), jnp.float32),
    )(p1, y1, dinv, b1p, W2p)


def _tc_f_body(p_ref, y2_ref, dinv_ref, b2_ref, batch_ref, gamma_ref,
               beta_ref, ow1_ref, ob1_ref, ow2_ref, ob2_ref,
               o_ref, h_ref, acc):
    i = pl.program_id(0)

    @pl.when(i == 0)
    def _():
        acc[...] = jnp.zeros_like(acc)

    tot = p_ref[0] + p_ref[1] - y2_ref[...]
    h2 = jnp.maximum(tot * dinv_ref[...] + b2_ref[...], 0.0)
    bt = batch_ref[0]
    gids = lax.broadcasted_iota(jnp.int32, (G, RB), 0)
    onehot = (gids == bt).astype(jnp.float32)
    acc[...] += jnp.dot(onehot, h2, preferred_element_type=jnp.float32, precision=lax.Precision.HIGHEST)

    @pl.when(i == pl.num_programs(0) - 1)
    def _():
        pooled = acc[...]
        h_ref[...] = pooled
        mean = jnp.mean(pooled, axis=0, keepdims=True)
        var = jnp.mean((pooled - mean) ** 2, axis=0, keepdims=True)
        xb = ((pooled - mean) / jnp.sqrt(var + 1e-5) * gamma_ref[...]
              + beta_ref[...])
        t = jnp.maximum(
            jnp.dot(xb, ow1_ref[...], preferred_element_type=jnp.float32, precision=lax.Precision.HIGHEST)
            + ob1_ref[...], 0.0)
        o_ref[...] = (jnp.dot(t, ow2_ref[...],
                              preferred_element_type=jnp.float32, precision=lax.Precision.HIGHEST)
                      + ob2_ref[...])


def _tc_f(p2, y2, dinv, b2, batch2d, gamma, beta, ow1, ob1, ow2, ob2):
    grid = N // RB
    return pl.pallas_call(
        _tc_f_body,
        grid=(grid,),
        in_specs=[
            pl.BlockSpec((NC, RB, F2), lambda i: (0, i, 0)),
            pl.BlockSpec((RB, F2), lambda i: (i, 0)),
            pl.BlockSpec((RB, 1), lambda i: (i, 0)),
            pl.BlockSpec((1, F2), lambda i: (0, 0)),
            pl.BlockSpec((1, 1, RB), lambda i: (i, 0, 0)),
            pl.BlockSpec((1, F2), lambda i: (0, 0)),
            pl.BlockSpec((1, F2), lambda i: (0, 0)),
            pl.BlockSpec((F2, 24), lambda i: (0, 0)),
            pl.BlockSpec((1, 24), lambda i: (0, 0)),
            pl.BlockSpec((24, 1), lambda i: (0, 0)),
            pl.BlockSpec((1, 1), lambda i: (0, 0)),
        ],
        out_specs=[
            pl.BlockSpec((G, 1), lambda i: (0, 0)),
            pl.BlockSpec((G, F2), lambda i: (0, 0)),
        ],
        out_shape=[
            jax.ShapeDtypeStruct((G, 1), jnp.float32),
            jax.ShapeDtypeStruct((G, F2), jnp.float32),
        ],
        scratch_shapes=[pltpu.VMEM((G, F2), jnp.float32)],
    )(p2, y2, dinv, b2, batch2d, gamma, beta, ow1, ob1, ow2, ob2)


def kernel(x, edge_index, batch, W1, b1, W2, b2, gamma, beta, ow1, ob1, ow2,
           ob2):
    src = edge_index[0].astype(jnp.int32)
    dst = edge_index[1].astype(jnp.int32)
    e = src.shape[0]
    pad = E_PAD - e
    src_p = jnp.concatenate([src, jnp.zeros((pad,), jnp.int32)])
    dst_p = jnp.concatenate([dst, jnp.full((pad,), N, jnp.int32)])
    src_p = src_p.reshape(NC, NS, CHUNKS, CH)
    dst_p = dst_p.reshape(NC, NS, CHUNKS, CH)

    zeros16 = jnp.zeros((N, 16), jnp.float32)
    ones_blk = jnp.ones((CH, 16), jnp.float32)
    degp = _deg_kernel(zeros16, ones_blk, dst_p)

    y1, dinv = _tc_b(degp, x, W1)
    p1 = _edge_pass(y1, src_p, dst_p, F1)

    b1p = jnp.pad(b1, (0, F1 - b1.shape[0])).reshape(1, F1)
    W2p = jnp.pad(W2, ((0, F1 - W2.shape[0]), (0, 0)))
    y2 = _tc_d(p1, y1, dinv, b1p, W2p)
    p2 = _edge_pass(y2, src_p, dst_p, F2)

    batch2d = batch.astype(jnp.int32).reshape(N // RB, 1, RB)
    o, h = _tc_f(p2, y2, dinv, b2.reshape(1, F2), batch2d,
                 gamma.reshape(1, F2), beta.reshape(1, F2), ow1,
                 ob1.reshape(1, 24), ow2, ob2.reshape(1, 1))
    return (o, h)


# speedup vs baseline: 45.5498x; 1.5073x over previous
import functools

import jax
import jax.numpy as jnp
from jax import lax
from jax.experimental import pallas as pl
from jax.experimental.pallas import tpu as pltpu
from jax.experimental.pallas import tpu_sc as plsc

N = 10000
G = 64
NC, NS = 2, 16
CH = 128
CHUNKS = 80
K = 8
E_PAD = NC * NS * CHUNKS * CH
ROWS_PER_TILE = N // NS
RB = 2000
F1 = 48
F2 = 16


def _sc_mesh():
    return plsc.VectorSubcoreMesh(core_axis_name="c", subcore_axis_name="s")


_SC_PARAMS = pltpu.CompilerParams(use_tc_tiling_on_sc=False)


def _bdot(a, b):
    ab = a.astype(jnp.bfloat16).astype(jnp.float32)
    bb = b.astype(jnp.bfloat16).astype(jnp.float32)
    return jnp.dot(ab, bb, preferred_element_type=jnp.float32,
                   precision=lax.Precision.HIGHEST)


def _deg_kernel(zeros_hbm, ones_hbm, dst_hbm):
    @functools.partial(
        pl.kernel,
        out_type=jax.ShapeDtypeStruct((NC, N, 16), jnp.float32),
        mesh=_sc_mesh(),
        compiler_params=_SC_PARAMS,
        scratch_types=[
            pltpu.VMEM((CHUNKS, CH), jnp.int32),
            pltpu.VMEM((CH, 16), jnp.float32),
            pltpu.VMEM_SHARED((N + 16, 16), jnp.float32),
            pltpu.SemaphoreType.DMA,
        ],
    )
    def k(zeros_h, ones_h, dst_h, out_h, dst_v, ones_v, accum, ssem):
        c = lax.axis_index("c")
        s = lax.axis_index("s")
        row0 = s * ROWS_PER_TILE
        pltpu.sync_copy(dst_h.at[c, s], dst_v)
        pltpu.sync_copy(ones_h, ones_v)
        pltpu.sync_copy(zeros_h.at[pl.ds(row0, ROWS_PER_TILE)],
                        accum.at[pl.ds(row0, ROWS_PER_TILE)])
        plsc.subcore_barrier()

        def body(g, carry):
            descs = [pltpu.async_copy(ones_v, accum.at[dst_v.at[g * K + b]],
                                      ssem, add=True) for b in range(K)]
            for d in descs:
                d.wait()
            return carry

        lax.fori_loop(0, CHUNKS // K, body, 0)
        plsc.subcore_barrier()
        pltpu.sync_copy(accum.at[pl.ds(row0, ROWS_PER_TILE)],
                        out_h.at[c, pl.ds(row0, ROWS_PER_TILE)])

    return k(zeros_hbm, ones_hbm, dst_hbm)


def _edge_pass(table_hbm, src_hbm, dst_hbm, F):
    @functools.partial(
        pl.kernel,
        out_type=jax.ShapeDtypeStruct((NC, N, F), jnp.float32),
        mesh=_sc_mesh(),
        compiler_params=_SC_PARAMS,
        scratch_types=[
            pltpu.VMEM((CHUNKS, CH), jnp.int32),
            pltpu.VMEM((CHUNKS, CH), jnp.int32),
            pltpu.VMEM((K, CH, F), jnp.float32),
            pltpu.VMEM_SHARED((N + 16, F), jnp.float32),
            pltpu.VMEM_SHARED((N + 16, F), jnp.float32),
            pltpu.SemaphoreType.DMA((K,)),
            pltpu.SemaphoreType.DMA((K,)),
        ],
    )
    def k(table_h, src_h, dst_h, out_h, src_v, dst_v, rows_v, accum, ytab,
          gsem, ssem):
        c = lax.axis_index("c")
        s = lax.axis_index("s")
        row0 = s * ROWS_PER_TILE
        pltpu.sync_copy(src_h.at[c, s], src_v)
        pltpu.sync_copy(dst_h.at[c, s], dst_v)
        pltpu.sync_copy(table_h.at[pl.ds(row0, ROWS_PER_TILE)],
                        accum.at[pl.ds(row0, ROWS_PER_TILE)])
        pltpu.sync_copy(table_h.at[pl.ds(row0, ROWS_PER_TILE)],
                        ytab.at[pl.ds(row0, ROWS_PER_TILE)])
        plsc.subcore_barrier()

        ng = CHUNKS // K

        def fire_gather(g, b):
            return pltpu.async_copy(ytab.at[src_v.at[g * K + b]],
                                    rows_v.at[b], gsem.at[b])

        def gather_desc(g, b):
            return pltpu.make_async_copy(ytab.at[src_v.at[g * K + b]],
                                         rows_v.at[b], gsem.at[b])

        def fire_scatter(g, b):
            return pltpu.async_copy(rows_v.at[b],
                                    accum.at[dst_v.at[g * K + b]],
                                    ssem.at[b], add=True)

        def scatter_desc(g, b):
            return pltpu.make_async_copy(rows_v.at[b],
                                         accum.at[dst_v.at[g * K + b]],
                                         ssem.at[b])

        for b in range(K):
            fire_gather(0, b)

        def body(g, carry):
            for b in range(K):
                gather_desc(g, b).wait()
                fire_scatter(g, b)
            for b in range(K):
                scatter_desc(g, b).wait()
                fire_gather(g + 1, b)
            return carry

        lax.fori_loop(0, ng - 1, body, 0)
        for b in range(K):
            gather_desc(ng - 1, b).wait()
            fire_scatter(ng - 1, b)
        for b in range(K):
            scatter_desc(ng - 1, b).wait()
        plsc.subcore_barrier()
        pltpu.sync_copy(accum.at[pl.ds(row0, ROWS_PER_TILE)],
                        out_h.at[c, pl.ds(row0, ROWS_PER_TILE)])

    return k(table_hbm, src_hbm, dst_hbm)


def _tc_b_body(degp_ref, x_ref, w1_ref, y1_ref, dinv_ref):
    deg = degp_ref[0, :, 0:1] + degp_ref[1, :, 0:1] + 1.0
    dinv = lax.rsqrt(deg)
    xw = _bdot(x_ref[...], w1_ref[...])
    y = dinv * xw
    y1_ref[...] = jnp.concatenate(
        [y, jnp.zeros((y.shape[0], F1 - y.shape[1]), jnp.float32)], axis=1)
    dinv_ref[...] = dinv


def _tc_b(degp, x, W1):
    grid = N // RB
    return pl.pallas_call(
        _tc_b_body,
        grid=(grid,),
        in_specs=[
            pl.BlockSpec((NC, RB, 16), lambda i: (0, i, 0)),
            pl.BlockSpec((RB, 128), lambda i: (i, 0)),
            pl.BlockSpec((128, 36), lambda i: (0, 0)),
        ],
        out_specs=[
            pl.BlockSpec((RB, F1), lambda i: (i, 0)),
            pl.BlockSpec((RB, 1), lambda i: (i, 0)),
        ],
        out_shape=[
            jax.ShapeDtypeStruct((N, F1), jnp.float32),
            jax.ShapeDtypeStruct((N, 1), jnp.float32),
        ],
    )(degp, x, W1)


def _tc_d_body(p_ref, y1_ref, dinv_ref, b1_ref, w2_ref, y2_ref):
    tot = p_ref[0] + p_ref[1] - y1_ref[...]
    h1 = jnp.maximum(tot * dinv_ref[...] + b1_ref[...], 0.0)
    xw2 = _bdot(h1, w2_ref[...])
    y2_ref[...] = dinv_ref[...] * xw2


def _tc_d(p1, y1, dinv, b1p, W2p):
    grid = N // RB
    return pl.pallas_call(
        _tc_d_body,
        grid=(grid,),
        in_specs=[
            pl.BlockSpec((NC, RB, F1), lambda i: (0, i, 0)),
            pl.BlockSpec((RB, F1), lambda i: (i, 0)),
            pl.BlockSpec((RB, 1), lambda i: (i, 0)),
            pl.BlockSpec((1, F1), lambda i: (0, 0)),
            pl.BlockSpec((F1, F2), lambda i: (0, 0)),
        ],
        out_specs=pl.BlockSpec((RB, F2), lambda i: (i, 0)),
        out_shape=jax.ShapeDtypeStruct((N, F2), jnp.float32),
    )(p1, y1, dinv, b1p, W2p)


def _tc_f_body(p_ref, y2_ref, dinv_ref, b2_ref, batch_ref, gamma_ref,
               beta_ref, ow1_ref, ob1_ref, ow2_ref, ob2_ref,
               o_ref, h_ref, acc):
    i = pl.program_id(0)

    @pl.when(i == 0)
    def _():
        acc[...] = jnp.zeros_like(acc)

    tot = p_ref[0] + p_ref[1] - y2_ref[...]
    h2 = jnp.maximum(tot * dinv_ref[...] + b2_ref[...], 0.0)
    bt = batch_ref[0]
    gids = lax.broadcasted_iota(jnp.int32, (G, RB), 0)
    onehot = (gids == bt).astype(jnp.float32)
    acc[...] += jnp.dot(onehot, h2, preferred_element_type=jnp.float32, precision=lax.Precision.HIGHEST)

    @pl.when(i == pl.num_programs(0) - 1)
    def _():
        pooled = acc[...]
        h_ref[...] = pooled
        mean = jnp.mean(pooled, axis=0, keepdims=True)
        var = jnp.mean((pooled - mean) ** 2, axis=0, keepdims=True)
        xb = ((pooled - mean) / jnp.sqrt(var + 1e-5) * gamma_ref[...]
              + beta_ref[...])
        t = jnp.maximum(_bdot(xb, ow1_ref[...]) + ob1_ref[...], 0.0)
        o_ref[...] = _bdot(t, ow2_ref[...]) + ob2_ref[...]


def _tc_f(p2, y2, dinv, b2, batch2d, gamma, beta, ow1, ob1, ow2, ob2):
    grid = N // RB
    return pl.pallas_call(
        _tc_f_body,
        grid=(grid,),
        in_specs=[
            pl.BlockSpec((NC, RB, F2), lambda i: (0, i, 0)),
            pl.BlockSpec((RB, F2), lambda i: (i, 0)),
            pl.BlockSpec((RB, 1), lambda i: (i, 0)),
            pl.BlockSpec((1, F2), lambda i: (0, 0)),
            pl.BlockSpec((1, 1, RB), lambda i: (i, 0, 0)),
            pl.BlockSpec((1, F2), lambda i: (0, 0)),
            pl.BlockSpec((1, F2), lambda i: (0, 0)),
            pl.BlockSpec((F2, 24), lambda i: (0, 0)),
            pl.BlockSpec((1, 24), lambda i: (0, 0)),
            pl.BlockSpec((24, 1), lambda i: (0, 0)),
            pl.BlockSpec((1, 1), lambda i: (0, 0)),
        ],
        out_specs=[
            pl.BlockSpec((G, 1), lambda i: (0, 0)),
            pl.BlockSpec((G, F2), lambda i: (0, 0)),
        ],
        out_shape=[
            jax.ShapeDtypeStruct((G, 1), jnp.float32),
            jax.ShapeDtypeStruct((G, F2), jnp.float32),
        ],
        scratch_shapes=[pltpu.VMEM((G, F2), jnp.float32)],
    )(p2, y2, dinv, b2, batch2d, gamma, beta, ow1, ob1, ow2, ob2)


def kernel(x, edge_index, batch, W1, b1, W2, b2, gamma, beta, ow1, ob1, ow2,
           ob2):
    src = edge_index[0].astype(jnp.int32)
    dst = edge_index[1].astype(jnp.int32)
    e = src.shape[0]
    pad = E_PAD - e
    src_p = jnp.concatenate([src, jnp.zeros((pad,), jnp.int32)])
    dst_p = jnp.concatenate([dst, jnp.full((pad,), N, jnp.int32)])
    src_p = src_p.reshape(NC, NS, CHUNKS, CH)
    dst_p = dst_p.reshape(NC, NS, CHUNKS, CH)

    zeros16 = jnp.zeros((N, 16), jnp.float32)
    ones_blk = jnp.ones((CH, 16), jnp.float32)
    degp = _deg_kernel(zeros16, ones_blk, dst_p)

    y1, dinv = _tc_b(degp, x, W1)
    p1 = _edge_pass(y1, src_p, dst_p, F1)

    b1p = jnp.pad(b1, (0, F1 - b1.shape[0])).reshape(1, F1)
    W2p = jnp.pad(W2, ((0, F1 - W2.shape[0]), (0, 0)))
    y2 = _tc_d(p1, y1, dinv, b1p, W2p)
    p2 = _edge_pass(y2, src_p, dst_p, F2)

    batch2d = batch.astype(jnp.int32).reshape(N // RB, 1, RB)
    o, h = _tc_f(p2, y2, dinv, b2.reshape(1, F2), batch2d,
                 gamma.reshape(1, F2), beta.reshape(1, F2), ow1,
                 ob1.reshape(1, 24), ow2, ob2.reshape(1, 1))
    return (o, h)


# speedup vs baseline: 47.8478x; 1.0505x over previous
import functools

import jax
import jax.numpy as jnp
from jax import lax
from jax.experimental import pallas as pl
from jax.experimental.pallas import tpu as pltpu
from jax.experimental.pallas import tpu_sc as plsc

N = 10000
G = 64
NC, NS = 2, 16
CH = 128
CHUNKS = 80
K = 8
E_PAD = NC * NS * CHUNKS * CH
ROWS_PER_TILE = N // NS
RB = 2000
F1 = 48
F2 = 16


def _sc_mesh():
    return plsc.VectorSubcoreMesh(core_axis_name="c", subcore_axis_name="s")


_SC_PARAMS = pltpu.CompilerParams(use_tc_tiling_on_sc=False)


def _bdot(a, b):
    ab = a.astype(jnp.bfloat16).astype(jnp.float32)
    bb = b.astype(jnp.bfloat16).astype(jnp.float32)
    return jnp.dot(ab, bb, preferred_element_type=jnp.float32)


def _deg_kernel(zeros_hbm, ones_hbm, ei_hbm):
    @functools.partial(
        pl.kernel,
        out_type=jax.ShapeDtypeStruct((NC, N, 16), jnp.float32),
        mesh=_sc_mesh(),
        compiler_params=_SC_PARAMS,
        scratch_types=[
            pltpu.VMEM((CHUNKS, CH), jnp.int32),
            pltpu.VMEM((CH, 16), jnp.float32),
            pltpu.VMEM_SHARED((N + 16, 16), jnp.float32),
            pltpu.SemaphoreType.DMA,
        ],
    )
    def k(zeros_h, ones_h, ei_h, out_h, dst_v, ones_v, accum, ssem):
        c = lax.axis_index("c")
        s = lax.axis_index("s")
        row0 = s * ROWS_PER_TILE
        pltpu.sync_copy(ei_h.at[1, c, s], dst_v)
        pltpu.sync_copy(ones_h, ones_v)
        pltpu.sync_copy(zeros_h.at[pl.ds(row0, ROWS_PER_TILE)],
                        accum.at[pl.ds(row0, ROWS_PER_TILE)])
        plsc.subcore_barrier()

        def body(g, carry):
            descs = [pltpu.async_copy(ones_v, accum.at[dst_v.at[g * K + b]],
                                      ssem, add=True) for b in range(K)]
            for d in descs:
                d.wait()
            return carry

        lax.fori_loop(0, CHUNKS // K, body, 0)
        plsc.subcore_barrier()
        pltpu.sync_copy(accum.at[pl.ds(row0, ROWS_PER_TILE)],
                        out_h.at[c, pl.ds(row0, ROWS_PER_TILE)])

    return k(zeros_hbm, ones_hbm, ei_hbm)


def _edge_pass(table_hbm, ei_hbm, F):
    @functools.partial(
        pl.kernel,
        out_type=jax.ShapeDtypeStruct((NC, N, F), jnp.float32),
        mesh=_sc_mesh(),
        compiler_params=_SC_PARAMS,
        scratch_types=[
            pltpu.VMEM((CHUNKS, CH), jnp.int32),
            pltpu.VMEM((CHUNKS, CH), jnp.int32),
            pltpu.VMEM((K, CH, F), jnp.float32),
            pltpu.VMEM_SHARED((N + 16, F), jnp.float32),
            pltpu.VMEM_SHARED((N + 16, F), jnp.float32),
            pltpu.SemaphoreType.DMA((K,)),
            pltpu.SemaphoreType.DMA((K,)),
        ],
    )
    def k(table_h, ei_h, out_h, src_v, dst_v, rows_v, accum, ytab,
          gsem, ssem):
        c = lax.axis_index("c")
        s = lax.axis_index("s")
        row0 = s * ROWS_PER_TILE
        pltpu.sync_copy(ei_h.at[0, c, s], src_v)
        pltpu.sync_copy(ei_h.at[1, c, s], dst_v)
        pltpu.sync_copy(table_h.at[pl.ds(row0, ROWS_PER_TILE)],
                        accum.at[pl.ds(row0, ROWS_PER_TILE)])
        pltpu.sync_copy(table_h.at[pl.ds(row0, ROWS_PER_TILE)],
                        ytab.at[pl.ds(row0, ROWS_PER_TILE)])
        plsc.subcore_barrier()

        ng = CHUNKS // K

        def fire_gather(g, b):
            return pltpu.async_copy(ytab.at[src_v.at[g * K + b]],
                                    rows_v.at[b], gsem.at[b])

        def gather_desc(g, b):
            return pltpu.make_async_copy(ytab.at[src_v.at[g * K + b]],
                                         rows_v.at[b], gsem.at[b])

        def fire_scatter(g, b):
            return pltpu.async_copy(rows_v.at[b],
                                    accum.at[dst_v.at[g * K + b]],
                                    ssem.at[b], add=True)

        def scatter_desc(g, b):
            return pltpu.make_async_copy(rows_v.at[b],
                                         accum.at[dst_v.at[g * K + b]],
                                         ssem.at[b])

        for b in range(K):
            fire_gather(0, b)

        def body(g, carry):
            for b in range(K):
                gather_desc(g, b).wait()
                fire_scatter(g, b)
            for b in range(K):
                scatter_desc(g, b).wait()
                fire_gather(g + 1, b)
            return carry

        lax.fori_loop(0, ng - 1, body, 0)
        for b in range(K):
            gather_desc(ng - 1, b).wait()
            fire_scatter(ng - 1, b)
        for b in range(K):
            scatter_desc(ng - 1, b).wait()
        plsc.subcore_barrier()
        pltpu.sync_copy(accum.at[pl.ds(row0, ROWS_PER_TILE)],
                        out_h.at[c, pl.ds(row0, ROWS_PER_TILE)])

    return k(table_hbm, ei_hbm)


def _tc_b_body(degp_ref, x_ref, w1_ref, y1_ref, dinv_ref):
    deg = degp_ref[0, :, 0:1] + degp_ref[1, :, 0:1] + 1.0
    dinv = lax.rsqrt(deg)
    xw = _bdot(x_ref[...], w1_ref[...])
    y = dinv * xw
    y1_ref[...] = jnp.concatenate(
        [y, jnp.zeros((y.shape[0], F1 - y.shape[1]), jnp.float32)], axis=1)
    dinv_ref[...] = dinv


def _tc_b(degp, x, W1):
    grid = N // RB
    return pl.pallas_call(
        _tc_b_body,
        grid=(grid,),
        in_specs=[
            pl.BlockSpec((NC, RB, 16), lambda i: (0, i, 0)),
            pl.BlockSpec((RB, 128), lambda i: (i, 0)),
            pl.BlockSpec((128, 36), lambda i: (0, 0)),
        ],
        out_specs=[
            pl.BlockSpec((RB, F1), lambda i: (i, 0)),
            pl.BlockSpec((RB, 1), lambda i: (i, 0)),
        ],
        out_shape=[
            jax.ShapeDtypeStruct((N, F1), jnp.float32),
            jax.ShapeDtypeStruct((N, 1), jnp.float32),
        ],
    )(degp, x, W1)


def _tc_d_body(p_ref, y1_ref, dinv_ref, b1_ref, w2_ref, y2_ref):
    tot = p_ref[0] + p_ref[1] - y1_ref[...]
    h1 = jnp.maximum(tot * dinv_ref[...] + b1_ref[...], 0.0)
    xw2 = _bdot(h1, w2_ref[...])
    y2_ref[...] = dinv_ref[...] * xw2


def _tc_d(p1, y1, dinv, b1p, W2p):
    grid = N // RB
    return pl.pallas_call(
        _tc_d_body,
        grid=(grid,),
        in_specs=[
            pl.BlockSpec((NC, RB, F1), lambda i: (0, i, 0)),
            pl.BlockSpec((RB, F1), lambda i: (i, 0)),
            pl.BlockSpec((RB, 1), lambda i: (i, 0)),
            pl.BlockSpec((1, F1), lambda i: (0, 0)),
            pl.BlockSpec((F1, F2), lambda i: (0, 0)),
        ],
        out_specs=pl.BlockSpec((RB, F2), lambda i: (i, 0)),
        out_shape=jax.ShapeDtypeStruct((N, F2), jnp.float32),
    )(p1, y1, dinv, b1p, W2p)


def _tc_f_body(p_ref, y2_ref, dinv_ref, b2_ref, batch_ref, gamma_ref,
               beta_ref, ow1_ref, ob1_ref, ow2_ref, ob2_ref,
               o_ref, h_ref, acc):
    i = pl.program_id(0)

    @pl.when(i == 0)
    def _():
        acc[...] = jnp.zeros_like(acc)

    tot = p_ref[0] + p_ref[1] - y2_ref[...]
    h2 = jnp.maximum(tot * dinv_ref[...] + b2_ref[...], 0.0)
    bt = batch_ref[0]
    gids = lax.broadcasted_iota(jnp.int32, (G, RB), 0)
    onehot = (gids == bt).astype(jnp.float32)
    acc[...] += jnp.dot(onehot, h2, preferred_element_type=jnp.float32, precision=lax.Precision.HIGHEST)

    @pl.when(i == pl.num_programs(0) - 1)
    def _():
        pooled = acc[...]
        h_ref[...] = pooled
        mean = jnp.mean(pooled, axis=0, keepdims=True)
        var = jnp.mean((pooled - mean) ** 2, axis=0, keepdims=True)
        xb = ((pooled - mean) / jnp.sqrt(var + 1e-5) * gamma_ref[...]
              + beta_ref[...])
        t = jnp.maximum(_bdot(xb, ow1_ref[...]) + ob1_ref[...], 0.0)
        o_ref[...] = _bdot(t, ow2_ref[...]) + ob2_ref[...]


def _tc_f(p2, y2, dinv, b2, batch2d, gamma, beta, ow1, ob1, ow2, ob2):
    grid = N // RB
    return pl.pallas_call(
        _tc_f_body,
        grid=(grid,),
        in_specs=[
            pl.BlockSpec((NC, RB, F2), lambda i: (0, i, 0)),
            pl.BlockSpec((RB, F2), lambda i: (i, 0)),
            pl.BlockSpec((RB, 1), lambda i: (i, 0)),
            pl.BlockSpec((1, F2), lambda i: (0, 0)),
            pl.BlockSpec((1, 1, RB), lambda i: (i, 0, 0)),
            pl.BlockSpec((1, F2), lambda i: (0, 0)),
            pl.BlockSpec((1, F2), lambda i: (0, 0)),
            pl.BlockSpec((F2, 24), lambda i: (0, 0)),
            pl.BlockSpec((1, 24), lambda i: (0, 0)),
            pl.BlockSpec((24, 1), lambda i: (0, 0)),
            pl.BlockSpec((1, 1), lambda i: (0, 0)),
        ],
        out_specs=[
            pl.BlockSpec((G, 1), lambda i: (0, 0)),
            pl.BlockSpec((G, F2), lambda i: (0, 0)),
        ],
        out_shape=[
            jax.ShapeDtypeStruct((G, 1), jnp.float32),
            jax.ShapeDtypeStruct((G, F2), jnp.float32),
        ],
        scratch_shapes=[pltpu.VMEM((G, F2), jnp.float32)],
    )(p2, y2, dinv, b2, batch2d, gamma, beta, ow1, ob1, ow2, ob2)


def kernel(x, edge_index, batch, W1, b1, W2, b2, gamma, beta, ow1, ob1, ow2,
           ob2):
    ei = edge_index.astype(jnp.int32)
    e = ei.shape[1]
    pad = E_PAD - e
    pad_blk = jnp.broadcast_to(
        jnp.array([[0], [N]], jnp.int32), (2, pad))
    ei_p = jnp.concatenate([ei, pad_blk], axis=1).reshape(
        2, NC, NS, CHUNKS, CH)

    zeros16 = jnp.zeros((N, 16), jnp.float32)
    ones_blk = jnp.ones((CH, 16), jnp.float32)
    degp = _deg_kernel(zeros16, ones_blk, ei_p)

    y1, dinv = _tc_b(degp, x, W1)
    p1 = _edge_pass(y1, ei_p, F1)

    b1p = jnp.pad(b1, (0, F1 - b1.shape[0])).reshape(1, F1)
    W2p = jnp.pad(W2, ((0, F1 - W2.shape[0]), (0, 0)))
    y2 = _tc_d(p1, y1, dinv, b1p, W2p)
    p2 = _edge_pass(y2, ei_p, F2)

    batch2d = batch.astype(jnp.int32).reshape(N // RB, 1, RB)
    o, h = _tc_f(p2, y2, dinv, b2.reshape(1, F2), batch2d,
                 gamma.reshape(1, F2), beta.reshape(1, F2), ow1,
                 ob1.reshape(1, 24), ow2, ob2.reshape(1, 1))
    return (o, h)


# speedup vs baseline: 48.3727x; 1.0110x over previous
import functools

import jax
import jax.numpy as jnp
from jax import lax
from jax.experimental import pallas as pl
from jax.experimental.pallas import tpu as pltpu
from jax.experimental.pallas import tpu_sc as plsc

N = 10000
G = 64
NC, NS = 2, 16
CH = 128
CHUNKS = 80
K = 8
E_PAD = NC * NS * CHUNKS * CH
ROWS_PER_TILE = N // NS
RB = 5000
F1 = 48
F2 = 16


def _sc_mesh():
    return plsc.VectorSubcoreMesh(core_axis_name="c", subcore_axis_name="s")


_SC_PARAMS = pltpu.CompilerParams(use_tc_tiling_on_sc=False)


def _bdot(a, b):
    ab = a.astype(jnp.bfloat16).astype(jnp.float32)
    bb = b.astype(jnp.bfloat16).astype(jnp.float32)
    return jnp.dot(ab, bb, preferred_element_type=jnp.float32)


def _deg_kernel(zeros_hbm, ones_hbm, ei_hbm):
    @functools.partial(
        pl.kernel,
        out_type=jax.ShapeDtypeStruct((NC, N, 16), jnp.float32),
        mesh=_sc_mesh(),
        compiler_params=_SC_PARAMS,
        scratch_types=[
            pltpu.VMEM((CHUNKS, CH), jnp.int32),
            pltpu.VMEM((CH, 16), jnp.float32),
            pltpu.VMEM_SHARED((N + 16, 16), jnp.float32),
            pltpu.SemaphoreType.DMA,
        ],
    )
    def k(zeros_h, ones_h, ei_h, out_h, dst_v, ones_v, accum, ssem):
        c = lax.axis_index("c")
        s = lax.axis_index("s")
        row0 = s * ROWS_PER_TILE
        pltpu.sync_copy(ei_h.at[1, c, s], dst_v)
        pltpu.sync_copy(ones_h, ones_v)
        pltpu.sync_copy(zeros_h.at[pl.ds(row0, ROWS_PER_TILE)],
                        accum.at[pl.ds(row0, ROWS_PER_TILE)])
        plsc.subcore_barrier()

        def body(g, carry):
            descs = [pltpu.async_copy(ones_v, accum.at[dst_v.at[g * K + b]],
                                      ssem, add=True) for b in range(K)]
            for d in descs:
                d.wait()
            return carry

        lax.fori_loop(0, CHUNKS // K, body, 0)
        plsc.subcore_barrier()
        pltpu.sync_copy(accum.at[pl.ds(row0, ROWS_PER_TILE)],
                        out_h.at[c, pl.ds(row0, ROWS_PER_TILE)])

    return k(zeros_hbm, ones_hbm, ei_hbm)


def _edge_pass(table_hbm, ei_hbm, F):
    @functools.partial(
        pl.kernel,
        out_type=jax.ShapeDtypeStruct((NC, N, F), jnp.float32),
        mesh=_sc_mesh(),
        compiler_params=_SC_PARAMS,
        scratch_types=[
            pltpu.VMEM((CHUNKS, CH), jnp.int32),
            pltpu.VMEM((CHUNKS, CH), jnp.int32),
            pltpu.VMEM((K, CH, F), jnp.float32),
            pltpu.VMEM_SHARED((N + 16, F), jnp.float32),
            pltpu.VMEM_SHARED((N + 16, F), jnp.float32),
            pltpu.SemaphoreType.DMA((K,)),
            pltpu.SemaphoreType.DMA((K,)),
        ],
    )
    def k(table_h, ei_h, out_h, src_v, dst_v, rows_v, accum, ytab,
          gsem, ssem):
        c = lax.axis_index("c")
        s = lax.axis_index("s")
        row0 = s * ROWS_PER_TILE
        pltpu.sync_copy(ei_h.at[0, c, s], src_v)
        pltpu.sync_copy(ei_h.at[1, c, s], dst_v)
        pltpu.sync_copy(table_h.at[pl.ds(row0, ROWS_PER_TILE)],
                        accum.at[pl.ds(row0, ROWS_PER_TILE)])
        pltpu.sync_copy(table_h.at[pl.ds(row0, ROWS_PER_TILE)],
                        ytab.at[pl.ds(row0, ROWS_PER_TILE)])
        plsc.subcore_barrier()

        ng = CHUNKS // K

        def fire_gather(g, b):
            return pltpu.async_copy(ytab.at[src_v.at[g * K + b]],
                                    rows_v.at[b], gsem.at[b])

        def gather_desc(g, b):
            return pltpu.make_async_copy(ytab.at[src_v.at[g * K + b]],
                                         rows_v.at[b], gsem.at[b])

        def fire_scatter(g, b):
            return pltpu.async_copy(rows_v.at[b],
                                    accum.at[dst_v.at[g * K + b]],
                                    ssem.at[b], add=True)

        def scatter_desc(g, b):
            return pltpu.make_async_copy(rows_v.at[b],
                                         accum.at[dst_v.at[g * K + b]],
                                         ssem.at[b])

        for b in range(K):
            fire_gather(0, b)

        def body(g, carry):
            for b in range(K):
                gather_desc(g, b).wait()
                fire_scatter(g, b)
            for b in range(K):
                scatter_desc(g, b).wait()
                fire_gather(g + 1, b)
            return carry

        lax.fori_loop(0, ng - 1, body, 0)
        for b in range(K):
            gather_desc(ng - 1, b).wait()
            fire_scatter(ng - 1, b)
        for b in range(K):
            scatter_desc(ng - 1, b).wait()
        plsc.subcore_barrier()
        pltpu.sync_copy(accum.at[pl.ds(row0, ROWS_PER_TILE)],
                        out_h.at[c, pl.ds(row0, ROWS_PER_TILE)])

    return k(table_hbm, ei_hbm)


def _tc_b_body(degp_ref, x_ref, w1_ref, y1_ref, dinv_ref):
    deg = degp_ref[0, :, 0:1] + degp_ref[1, :, 0:1] + 1.0
    dinv = lax.rsqrt(deg)
    xw = _bdot(x_ref[...], w1_ref[...])
    y = dinv * xw
    y1_ref[...] = jnp.concatenate(
        [y, jnp.zeros((y.shape[0], F1 - y.shape[1]), jnp.float32)], axis=1)
    dinv_ref[...] = dinv


def _tc_b(degp, x, W1):
    grid = N // RB
    return pl.pallas_call(
        _tc_b_body,
        grid=(grid,),
        in_specs=[
            pl.BlockSpec((NC, RB, 16), lambda i: (0, i, 0)),
            pl.BlockSpec((RB, 128), lambda i: (i, 0)),
            pl.BlockSpec((128, 36), lambda i: (0, 0)),
        ],
        out_specs=[
            pl.BlockSpec((RB, F1), lambda i: (i, 0)),
            pl.BlockSpec((RB, 1), lambda i: (i, 0)),
        ],
        out_shape=[
            jax.ShapeDtypeStruct((N, F1), jnp.float32),
            jax.ShapeDtypeStruct((N, 1), jnp.float32),
        ],
    )(degp, x, W1)


def _tc_d_body(p_ref, y1_ref, dinv_ref, b1_ref, w2_ref, y2_ref):
    tot = p_ref[0] + p_ref[1] - y1_ref[...]
    h1 = jnp.maximum(tot * dinv_ref[...] + b1_ref[...], 0.0)
    xw2 = _bdot(h1, w2_ref[...])
    y2_ref[...] = dinv_ref[...] * xw2


def _tc_d(p1, y1, dinv, b1p, W2p):
    grid = N // RB
    return pl.pallas_call(
        _tc_d_body,
        grid=(grid,),
        in_specs=[
            pl.BlockSpec((NC, RB, F1), lambda i: (0, i, 0)),
            pl.BlockSpec((RB, F1), lambda i: (i, 0)),
            pl.BlockSpec((RB, 1), lambda i: (i, 0)),
            pl.BlockSpec((1, F1), lambda i: (0, 0)),
            pl.BlockSpec((F1, F2), lambda i: (0, 0)),
        ],
        out_specs=pl.BlockSpec((RB, F2), lambda i: (i, 0)),
        out_shape=jax.ShapeDtypeStruct((N, F2), jnp.float32),
    )(p1, y1, dinv, b1p, W2p)


def _tc_f_body(p_ref, y2_ref, dinv_ref, b2_ref, batch_ref, gamma_ref,
               beta_ref, ow1_ref, ob1_ref, ow2_ref, ob2_ref,
               o_ref, h_ref, acc):
    i = pl.program_id(0)

    @pl.when(i == 0)
    def _():
        acc[...] = jnp.zeros_like(acc)

    tot = p_ref[0] + p_ref[1] - y2_ref[...]
    h2 = jnp.maximum(tot * dinv_ref[...] + b2_ref[...], 0.0)
    bt = batch_ref[0]
    gids = lax.broadcasted_iota(jnp.int32, (G, RB), 0)
    onehot = (gids == bt).astype(jnp.float32)
    acc[...] += jnp.dot(onehot, h2, preferred_element_type=jnp.float32, precision=lax.Precision.HIGHEST)

    @pl.when(i == pl.num_programs(0) - 1)
    def _():
        pooled = acc[...]
        h_ref[...] = pooled
        mean = jnp.mean(pooled, axis=0, keepdims=True)
        var = jnp.mean((pooled - mean) ** 2, axis=0, keepdims=True)
        xb = ((pooled - mean) / jnp.sqrt(var + 1e-5) * gamma_ref[...]
              + beta_ref[...])
        t = jnp.maximum(_bdot(xb, ow1_ref[...]) + ob1_ref[...], 0.0)
        o_ref[...] = _bdot(t, ow2_ref[...]) + ob2_ref[...]


def _tc_f(p2, y2, dinv, b2, batch2d, gamma, beta, ow1, ob1, ow2, ob2):
    grid = N // RB
    return pl.pallas_call(
        _tc_f_body,
        grid=(grid,),
        in_specs=[
            pl.BlockSpec((NC, RB, F2), lambda i: (0, i, 0)),
            pl.BlockSpec((RB, F2), lambda i: (i, 0)),
            pl.BlockSpec((RB, 1), lambda i: (i, 0)),
            pl.BlockSpec((1, F2), lambda i: (0, 0)),
            pl.BlockSpec((1, 1, RB), lambda i: (i, 0, 0)),
            pl.BlockSpec((1, F2), lambda i: (0, 0)),
            pl.BlockSpec((1, F2), lambda i: (0, 0)),
            pl.BlockSpec((F2, 24), lambda i: (0, 0)),
            pl.BlockSpec((1, 24), lambda i: (0, 0)),
            pl.BlockSpec((24, 1), lambda i: (0, 0)),
            pl.BlockSpec((1, 1), lambda i: (0, 0)),
        ],
        out_specs=[
            pl.BlockSpec((G, 1), lambda i: (0, 0)),
            pl.BlockSpec((G, F2), lambda i: (0, 0)),
        ],
        out_shape=[
            jax.ShapeDtypeStruct((G, 1), jnp.float32),
            jax.ShapeDtypeStruct((G, F2), jnp.float32),
        ],
        scratch_shapes=[pltpu.VMEM((G, F2), jnp.float32)],
    )(p2, y2, dinv, b2, batch2d, gamma, beta, ow1, ob1, ow2, ob2)


def kernel(x, edge_index, batch, W1, b1, W2, b2, gamma, beta, ow1, ob1, ow2,
           ob2):
    ei = edge_index.astype(jnp.int32)
    e = ei.shape[1]
    pad = E_PAD - e
    pad_blk = jnp.broadcast_to(
        jnp.array([[0], [N]], jnp.int32), (2, pad))
    ei_p = jnp.concatenate([ei, pad_blk], axis=1).reshape(
        2, NC, NS, CHUNKS, CH)

    zeros16 = jnp.zeros((N, 16), jnp.float32)
    ones_blk = jnp.ones((CH, 16), jnp.float32)
    degp = _deg_kernel(zeros16, ones_blk, ei_p)

    y1, dinv = _tc_b(degp, x, W1)
    p1 = _edge_pass(y1, ei_p, F1)

    b1p = jnp.pad(b1, (0, F1 - b1.shape[0])).reshape(1, F1)
    W2p = jnp.pad(W2, ((0, F1 - W2.shape[0]), (0, 0)))
    y2 = _tc_d(p1, y1, dinv, b1p, W2p)
    p2 = _edge_pass(y2, ei_p, F2)

    batch2d = batch.astype(jnp.int32).reshape(N // RB, 1, RB)
    o, h = _tc_f(p2, y2, dinv, b2.reshape(1, F2), batch2d,
                 gamma.reshape(1, F2), beta.reshape(1, F2), ow1,
                 ob1.reshape(1, 24), ow2, ob2.reshape(1, 1))
    return (o, h)
